# Initial kernel scaffold; baseline (speedup 1.0000x reference)
import jax
import jax.numpy as jnp
from jax.experimental import pallas as pl


def kernel(inputs, word_table, pos_table, W, b, gamma, beta, moving_mean, moving_var):
    raise NotImplementedError("write your pallas kernel here")


# speedup vs baseline: 2.2342x; 2.2342x over previous
import functools

import jax
import jax.numpy as jnp
from jax import lax
from jax.experimental import pallas as pl
from jax.experimental.pallas import tpu as pltpu
from jax.experimental.pallas import tpu_sc as plsc

HID = 128
POS = 100000
BATCH = 1024
SEQ = 200

_ROWS = 2000
_GRID = POS // _ROWS


def _transform_body(pos_ref, word_ref, w_ref, scale_ref, bias2_ref, t2_ref, const_ref):
    ws = w_ref[...] * scale_ref[...]
    t2_ref[...] = jnp.dot(pos_ref[...], ws, preferred_element_type=jnp.float32)

    @pl.when(pl.program_id(0) == 0)
    def _():
        const_ref[...] = (
            jnp.dot(word_ref[...], ws, preferred_element_type=jnp.float32)
            + bias2_ref[...]
        )


def _transform(pos_table, word_s, W, scale, bias2):
    return pl.pallas_call(
        _transform_body,
        grid=(_GRID,),
        in_specs=[
            pl.BlockSpec((_ROWS, HID), lambda i: (i, 0)),
            pl.BlockSpec((SEQ, HID), lambda i: (0, 0)),
            pl.BlockSpec((HID, HID), lambda i: (0, 0)),
            pl.BlockSpec((1, HID), lambda i: (0, 0)),
            pl.BlockSpec((1, HID), lambda i: (0, 0)),
        ],
        out_specs=[
            pl.BlockSpec((_ROWS, HID), lambda i: (i, 0)),
            pl.BlockSpec((SEQ, HID), lambda i: (0, 0)),
        ],
        out_shape=[
            jax.ShapeDtypeStruct((POS, HID), jnp.float32),
            jax.ShapeDtypeStruct((SEQ, HID), jnp.float32),
        ],
    )(pos_table, word_s, W, scale, bias2)


_NW = 32
_CPW = BATCH // _NW
_HS = SEQ // 2


def _gather_body(table_hbm, idx_hbm, const_hbm, out_hbm, idx_v, rows_v, const_v, sem):
    wid = lax.axis_index("s") * 2 + lax.axis_index("c")
    pltpu.sync_copy(const_hbm, const_v)

    def chunk_body(j, carry):
        c = wid * _CPW + j
        pltpu.sync_copy(idx_hbm.at[c], idx_v)
        cp0 = pltpu.async_copy(table_hbm.at[idx_v.at[0]], rows_v.at[0], sem)
        cp1 = pltpu.async_copy(table_hbm.at[idx_v.at[1]], rows_v.at[1], sem)
        cp0.wait()
        cp1.wait()

        def add_row(t, carry2):
            h = t // _HS
            i = t % _HS
            for l in range(HID // 16):
                sl = (h, i, pl.ds(l * 16, 16))
                rows_v[sl] = rows_v[sl] + const_v[sl]
            return carry2

        lax.fori_loop(0, SEQ, add_row, 0)
        pltpu.sync_copy(rows_v, out_hbm.at[c])
        return carry

    lax.fori_loop(0, _CPW, chunk_body, 0)


def _gather(table2, idx3, const3):
    mesh = plsc.VectorSubcoreMesh(core_axis_name="c", subcore_axis_name="s")
    k = functools.partial(
        pl.kernel,
        mesh=mesh,
        out_type=jax.ShapeDtypeStruct((BATCH, 2, _HS, HID), jnp.float32),
        scratch_types=[
            pltpu.VMEM((2, _HS), jnp.int32),
            pltpu.VMEM((2, _HS, HID), jnp.float32),
            pltpu.VMEM((2, _HS, HID), jnp.float32),
            pltpu.SemaphoreType.DMA,
        ],
    )(_gather_body)
    return k(table2, idx3, const3)


def kernel(inputs, word_table, pos_table, W, b, gamma, beta, moving_mean, moving_var):
    scale = gamma * lax.rsqrt(moving_var + 1e-3)
    bias2 = (b - moving_mean) * scale + beta
    table2, const = _transform(
        pos_table, word_table[:SEQ], W, scale[None, :], bias2[None, :]
    )
    idx3 = inputs.reshape(BATCH, 2, _HS).astype(jnp.int32)
    const3 = const.reshape(2, _HS, HID)
    out4 = _gather(table2, idx3, const3)
    return out4.reshape(BATCH, SEQ, HID)


# speedup vs baseline: 2.9979x; 1.3419x over previous
import functools

import jax
import jax.numpy as jnp
from jax import lax
from jax.experimental import pallas as pl
from jax.experimental.pallas import tpu as pltpu
from jax.experimental.pallas import tpu_sc as plsc

HID = 128
POS = 100000
BATCH = 1024
SEQ = 200
NFLAT = BATCH * SEQ

_ROWS = 4000
_GRID = POS // _ROWS


def _transform_body(pos_ref, word_ref, w_ref, scale_ref, bias2_ref, t2_ref, const_ref):
    ws = w_ref[...] * scale_ref[...]
    t2_ref[...] = jnp.dot(pos_ref[...], ws, preferred_element_type=jnp.float32)

    @pl.when(pl.program_id(0) == 0)
    def _():
        const_ref[...] = (
            jnp.dot(word_ref[...], ws, preferred_element_type=jnp.float32)
            + bias2_ref[...]
        )


def _transform(pos_table, word_s, W, scale, bias2):
    return pl.pallas_call(
        _transform_body,
        grid=(_GRID,),
        in_specs=[
            pl.BlockSpec((_ROWS, HID), lambda i: (i, 0)),
            pl.BlockSpec((SEQ, HID), lambda i: (0, 0)),
            pl.BlockSpec((HID, HID), lambda i: (0, 0)),
            pl.BlockSpec((1, HID), lambda i: (0, 0)),
            pl.BlockSpec((1, HID), lambda i: (0, 0)),
        ],
        out_specs=[
            pl.BlockSpec((_ROWS, HID), lambda i: (i, 0)),
            pl.BlockSpec((SEQ, HID), lambda i: (0, 0)),
        ],
        out_shape=[
            jax.ShapeDtypeStruct((POS, HID), jnp.float32),
            jax.ShapeDtypeStruct((SEQ, HID), jnp.float32),
        ],
    )(pos_table, word_s, W, scale, bias2)


_NW = 32
_CHUNK = 256
_NCHUNK = NFLAT // _CHUNK
_CPW = _NCHUNK // _NW


def _gather_body(table_hbm, idx_hbm, const_hbm, out_hbm,
                 idx_a, idx_b, rows_a, rows_b, const_v,
                 gsem_a, gsem_b, wsem_a, wsem_b):
    wid = lax.axis_index("s") * 2 + lax.axis_index("c")
    pltpu.sync_copy(const_hbm, const_v)

    idx_bufs = (idx_a, idx_b)
    row_bufs = (rows_a, rows_b)
    gsems = (gsem_a, gsem_b)
    wsems = (wsem_a, wsem_b)

    def fire_gather(j, buf):
        c = wid * _CPW + j
        pltpu.sync_copy(idx_hbm.at[pl.ds(2 * c, 2)], idx_bufs[buf])
        cps = []
        for h in range(2):
            cps.append(pltpu.async_copy(
                table_hbm.at[idx_bufs[buf].at[h]], row_bufs[buf].at[h],
                gsems[buf]))
        return cps

    def add_const(j, buf):
        c = wid * _CPW + j
        rows = row_bufs[buf]

        def add_row(t, carry):
            s = lax.rem(c * _CHUNK + t, SEQ)
            h = t // 128
            i = lax.rem(t, 128)
            for l in range(HID // 16):
                plsc.addupdate(rows.at[h, i, pl.ds(l * 16, 16)],
                               const_v[s, pl.ds(l * 16, 16)])
            return carry

        lax.fori_loop(0, _CHUNK, add_row, 0)

    g_pend = {0: fire_gather(0, 0)}
    w_pend = {}
    for j in range(_CPW):
        buf = j & 1
        nxt = buf ^ 1
        if j + 1 < _CPW:
            if (j - 1) in w_pend:
                w_pend.pop(j - 1).wait()
            g_pend[j + 1] = fire_gather(j + 1, nxt)
        for cp in g_pend.pop(j):
            cp.wait()
        add_const(j, buf)
        c = wid * _CPW + j
        w_pend[j] = pltpu.async_copy(row_bufs[buf], out_hbm.at[c], wsems[buf])
    for d in w_pend.values():
        d.wait()


def _gather(table2, idx2, const2):
    mesh = plsc.VectorSubcoreMesh(core_axis_name="c", subcore_axis_name="s")
    k = functools.partial(
        pl.kernel,
        mesh=mesh,
        out_type=jax.ShapeDtypeStruct((_NCHUNK, 2, 128, HID), jnp.float32),
        scratch_types=[
            pltpu.VMEM((2, 128), jnp.int32),
            pltpu.VMEM((2, 128), jnp.int32),
            pltpu.VMEM((2, 128, HID), jnp.float32),
            pltpu.VMEM((2, 128, HID), jnp.float32),
            pltpu.VMEM((SEQ, HID), jnp.float32),
            pltpu.SemaphoreType.DMA,
            pltpu.SemaphoreType.DMA,
            pltpu.SemaphoreType.DMA,
            pltpu.SemaphoreType.DMA,
        ],
    )(_gather_body)
    return k(table2, idx2, const2)


def kernel(inputs, word_table, pos_table, W, b, gamma, beta, moving_mean, moving_var):
    scale = gamma * lax.rsqrt(moving_var + 1e-3)
    bias2 = (b - moving_mean) * scale + beta
    table2, const = _transform(
        pos_table, word_table[:SEQ], W, scale[None, :], bias2[None, :]
    )
    idx2 = inputs.reshape(NFLAT // 128, 128).astype(jnp.int32)
    out4 = _gather(table2, idx2, const)
    return out4.reshape(BATCH, SEQ, HID)


# speedup vs baseline: 4.9448x; 1.6494x over previous
import functools

import jax
import jax.numpy as jnp
from jax import lax
from jax.experimental import pallas as pl
from jax.experimental.pallas import tpu as pltpu
from jax.experimental.pallas import tpu_sc as plsc

HID = 128
POS = 100000
BATCH = 1024
SEQ = 200
NFLAT = BATCH * SEQ

_ROWS = 4000
_GRID = POS // _ROWS


_CEXT = 456


def _transform_body(pos_ref, word_ref, w_ref, scale_ref, bias2_ref, t2_ref, const_ref):
    ws = w_ref[...] * scale_ref[...]
    t2_ref[...] = jnp.dot(pos_ref[...], ws, preferred_element_type=jnp.float32)

    @pl.when(pl.program_id(0) == 0)
    def _():
        cmat = (
            jnp.dot(word_ref[...], ws, preferred_element_type=jnp.float32)
            + bias2_ref[...]
        )
        const_ref[0:SEQ, :] = cmat
        const_ref[SEQ : 2 * SEQ, :] = cmat
        const_ref[2 * SEQ : _CEXT, :] = cmat[: _CEXT - 2 * SEQ, :]


def _transform(pos_table, word_s, W, scale, bias2):
    return pl.pallas_call(
        _transform_body,
        grid=(_GRID,),
        in_specs=[
            pl.BlockSpec((_ROWS, HID), lambda i: (i, 0)),
            pl.BlockSpec((SEQ, HID), lambda i: (0, 0)),
            pl.BlockSpec((HID, HID), lambda i: (0, 0)),
            pl.BlockSpec((1, HID), lambda i: (0, 0)),
            pl.BlockSpec((1, HID), lambda i: (0, 0)),
        ],
        out_specs=[
            pl.BlockSpec((_ROWS, HID), lambda i: (i, 0)),
            pl.BlockSpec((_CEXT, HID), lambda i: (0, 0)),
        ],
        out_shape=[
            jax.ShapeDtypeStruct((POS, HID), jnp.float32),
            jax.ShapeDtypeStruct((_CEXT, HID), jnp.float32),
        ],
    )(pos_table, word_s, W, scale, bias2)


_NW = 32
_CHUNK = 256
_NCHUNK = NFLAT // _CHUNK
_CPW = _NCHUNK // _NW


def _gather_body(table_hbm, idx_hbm, const_hbm, out_hbm,
                 idx_a, idx_b, rows_a, rows_b, const_v,
                 gsem_a, gsem_b, wsem_a, wsem_b):
    wid = lax.axis_index("s") * 2 + lax.axis_index("c")
    pltpu.sync_copy(const_hbm, const_v)

    idx_bufs = (idx_a, idx_b)
    row_bufs = (rows_a, rows_b)
    gsems = (gsem_a, gsem_b)
    wsems = (wsem_a, wsem_b)

    def fire_gather(j, buf):
        c = wid * _CPW + j
        pltpu.sync_copy(idx_hbm.at[pl.ds(2 * c, 2)], idx_bufs[buf])
        cps = []
        for h in range(2):
            cps.append(pltpu.async_copy(
                table_hbm.at[idx_bufs[buf].at[h]],
                row_bufs[buf].at[pl.ds(h * 128, 128)],
                gsems[buf]))
        return cps

    def add_const(j, buf):
        c = wid * _CPW + j
        rows = row_bufs[buf]
        s0 = lax.rem(c * _CHUNK, SEQ)

        def add_row(t, carry):
            vals = [const_v[s0 + t, pl.ds(l * 16, 16)] for l in range(HID // 16)]
            for l in range(HID // 16):
                plsc.addupdate(rows.at[t, pl.ds(l * 16, 16)], vals[l])
            return carry

        lax.fori_loop(0, _CHUNK, add_row, 0)

    g_pend = {0: fire_gather(0, 0)}
    w_pend = {}
    for j in range(_CPW):
        buf = j & 1
        nxt = buf ^ 1
        if j + 1 < _CPW:
            if (j - 1) in w_pend:
                w_pend.pop(j - 1).wait()
            g_pend[j + 1] = fire_gather(j + 1, nxt)
        for cp in g_pend.pop(j):
            cp.wait()
        add_const(j, buf)
        c = wid * _CPW + j
        w_pend[j] = pltpu.async_copy(
            row_bufs[buf], out_hbm.at[pl.ds(c * _CHUNK, _CHUNK)], wsems[buf])
    for d in w_pend.values():
        d.wait()


def _gather(table2, idx2, const2):
    mesh = plsc.VectorSubcoreMesh(core_axis_name="c", subcore_axis_name="s")
    k = functools.partial(
        pl.kernel,
        mesh=mesh,
        out_type=jax.ShapeDtypeStruct((NFLAT, HID), jnp.float32),
        scratch_types=[
            pltpu.VMEM((2, 128), jnp.int32),
            pltpu.VMEM((2, 128), jnp.int32),
            pltpu.VMEM((_CHUNK, HID), jnp.float32),
            pltpu.VMEM((_CHUNK, HID), jnp.float32),
            pltpu.VMEM((_CEXT, HID), jnp.float32),
            pltpu.SemaphoreType.DMA,
            pltpu.SemaphoreType.DMA,
            pltpu.SemaphoreType.DMA,
            pltpu.SemaphoreType.DMA,
        ],
    )(_gather_body)
    return k(table2, idx2, const2)


def kernel(inputs, word_table, pos_table, W, b, gamma, beta, moving_mean, moving_var):
    scale = gamma * lax.rsqrt(moving_var + 1e-3)
    bias2 = (b - moving_mean) * scale + beta
    table2, const = _transform(
        pos_table, word_table[:SEQ], W, scale[None, :], bias2[None, :]
    )
    idx2 = inputs.reshape(NFLAT // 128, 128).astype(jnp.int32)
    out4 = _gather(table2, idx2, const)
    return out4.reshape(BATCH, SEQ, HID)


# speedup vs baseline: 5.1195x; 1.0353x over previous
import functools

import jax
import jax.numpy as jnp
from jax import lax
from jax.experimental import pallas as pl
from jax.experimental.pallas import tpu as pltpu
from jax.experimental.pallas import tpu_sc as plsc

HID = 128
POS = 100000
BATCH = 1024
SEQ = 200
NFLAT = BATCH * SEQ

_ROWS = 10000
_GRID = POS // _ROWS


_CEXT = 456


def _transform_body(pos_ref, word_ref, w_ref, scale_ref, bias2_ref, t2_ref, const_ref):
    ws = w_ref[...] * scale_ref[...]
    t2_ref[...] = jnp.dot(pos_ref[...], ws, preferred_element_type=jnp.float32)

    @pl.when(pl.program_id(0) == 0)
    def _():
        cmat = (
            jnp.dot(word_ref[...], ws, preferred_element_type=jnp.float32)
            + bias2_ref[...]
        )
        const_ref[0:SEQ, :] = cmat
        const_ref[SEQ : 2 * SEQ, :] = cmat
        const_ref[2 * SEQ : _CEXT, :] = cmat[: _CEXT - 2 * SEQ, :]


def _transform(pos_table, word_s, W, scale, bias2):
    return pl.pallas_call(
        _transform_body,
        grid=(_GRID,),
        in_specs=[
            pl.BlockSpec((_ROWS, HID), lambda i: (i, 0)),
            pl.BlockSpec((SEQ, HID), lambda i: (0, 0)),
            pl.BlockSpec((HID, HID), lambda i: (0, 0)),
            pl.BlockSpec((1, HID), lambda i: (0, 0)),
            pl.BlockSpec((1, HID), lambda i: (0, 0)),
        ],
        out_specs=[
            pl.BlockSpec((_ROWS, HID), lambda i: (i, 0)),
            pl.BlockSpec((_CEXT, HID), lambda i: (0, 0)),
        ],
        out_shape=[
            jax.ShapeDtypeStruct((POS, HID), jnp.float32),
            jax.ShapeDtypeStruct((_CEXT, HID), jnp.float32),
        ],
    )(pos_table, word_s, W, scale, bias2)


_NW = 32
_CHUNK = 256
_NCHUNK = NFLAT // _CHUNK
_CPW = _NCHUNK // _NW


def _gather_body(table_hbm, idx_hbm, const_hbm, out_hbm,
                 idx_all, rows_a, rows_b, const_v,
                 gsem_a, gsem_b, wsem_a, wsem_b):
    wid = lax.axis_index("s") * 2 + lax.axis_index("c")
    pltpu.sync_copy(const_hbm, const_v)

    row_bufs = (rows_a, rows_b)
    gsems = (gsem_a, gsem_b)
    wsems = (wsem_a, wsem_b)

    def fire_gather(j, buf):
        c = wid * _CPW + j
        pltpu.sync_copy(idx_hbm.at[pl.ds(2 * c, 2)], idx_all.at[buf])
        cps = []
        for h in range(2):
            cps.append(pltpu.async_copy(
                table_hbm.at[idx_all.at[buf].at[h]],
                row_bufs[buf].at[pl.ds(h * 128, 128)],
                gsems[buf]))
        return cps

    def add_const(j, buf):
        c = wid * _CPW + j
        rows = row_bufs[buf]
        s0 = lax.rem(c * _CHUNK, SEQ)

        def add_row(t, carry):
            vals = [const_v[s0 + t, pl.ds(l * 16, 16)] for l in range(HID // 16)]
            for l in range(HID // 16):
                plsc.addupdate(rows.at[t, pl.ds(l * 16, 16)], vals[l])
            return carry

        lax.fori_loop(0, _CHUNK, add_row, 0)

    g_pend = {0: fire_gather(0, 0)}
    w_pend = {}
    for j in range(_CPW):
        buf = j & 1
        nxt = buf ^ 1
        if j + 1 < _CPW:
            if (j - 1) in w_pend:
                w_pend.pop(j - 1).wait()
            g_pend[j + 1] = fire_gather(j + 1, nxt)
        for cp in g_pend.pop(j):
            cp.wait()
        add_const(j, buf)
        c = wid * _CPW + j
        w_pend[j] = pltpu.async_copy(
            row_bufs[buf], out_hbm.at[pl.ds(c * _CHUNK, _CHUNK)], wsems[buf])
    for d in w_pend.values():
        d.wait()


def _gather(table2, idx2, const2):
    mesh = plsc.VectorSubcoreMesh(core_axis_name="c", subcore_axis_name="s")
    k = functools.partial(
        pl.kernel,
        mesh=mesh,
        out_type=jax.ShapeDtypeStruct((NFLAT, HID), jnp.float32),
        scratch_types=[
            pltpu.VMEM((2, 2, 128), jnp.int32),
            pltpu.VMEM((_CHUNK, HID), jnp.float32),
            pltpu.VMEM((_CHUNK, HID), jnp.float32),
            pltpu.VMEM((_CEXT, HID), jnp.float32),
            pltpu.SemaphoreType.DMA,
            pltpu.SemaphoreType.DMA,
            pltpu.SemaphoreType.DMA,
            pltpu.SemaphoreType.DMA,
        ],
    )(_gather_body)
    return k(table2, idx2, const2)


def kernel(inputs, word_table, pos_table, W, b, gamma, beta, moving_mean, moving_var):
    scale = gamma * lax.rsqrt(moving_var + 1e-3)
    bias2 = (b - moving_mean) * scale + beta
    table2, const = _transform(
        pos_table, word_table[:SEQ], W, scale[None, :], bias2[None, :]
    )
    idx2 = inputs.reshape(NFLAT // 128, 128).astype(jnp.int32)
    out4 = _gather(table2, idx2, const)
    return out4.reshape(BATCH, SEQ, HID)


# speedup vs baseline: 5.1868x; 1.0132x over previous
import functools

import jax
import jax.numpy as jnp
from jax import lax
from jax.experimental import pallas as pl
from jax.experimental.pallas import tpu as pltpu
from jax.experimental.pallas import tpu_sc as plsc

HID = 128
POS = 100000
BATCH = 1024
SEQ = 200
NFLAT = BATCH * SEQ

_ROWS = 20000
_GRID = POS // _ROWS


_CEXT = 456


def _transform_body(pos_ref, word_ref, w_ref, scale_ref, bias2_ref, t2_ref, const_ref):
    ws = w_ref[...] * scale_ref[...]
    t2_ref[...] = jnp.dot(pos_ref[...], ws, preferred_element_type=jnp.float32)

    @pl.when(pl.program_id(0) == 0)
    def _():
        cmat = (
            jnp.dot(word_ref[...], ws, preferred_element_type=jnp.float32)
            + bias2_ref[...]
        )
        const_ref[0:SEQ, :] = cmat
        const_ref[SEQ : 2 * SEQ, :] = cmat
        const_ref[2 * SEQ : _CEXT, :] = cmat[: _CEXT - 2 * SEQ, :]


def _transform(pos_table, word_s, W, scale, bias2):
    return pl.pallas_call(
        _transform_body,
        grid=(_GRID,),
        in_specs=[
            pl.BlockSpec((_ROWS, HID), lambda i: (i, 0)),
            pl.BlockSpec((SEQ, HID), lambda i: (0, 0)),
            pl.BlockSpec((HID, HID), lambda i: (0, 0)),
            pl.BlockSpec((1, HID), lambda i: (0, 0)),
            pl.BlockSpec((1, HID), lambda i: (0, 0)),
        ],
        out_specs=[
            pl.BlockSpec((_ROWS, HID), lambda i: (i, 0)),
            pl.BlockSpec((_CEXT, HID), lambda i: (0, 0)),
        ],
        out_shape=[
            jax.ShapeDtypeStruct((POS, HID), jnp.float32),
            jax.ShapeDtypeStruct((_CEXT, HID), jnp.float32),
        ],
    )(pos_table, word_s, W, scale, bias2)


_NW = 32
_CHUNK = 256
_NCHUNK = NFLAT // _CHUNK
_CPW = _NCHUNK // _NW


def _gather_body(table_hbm, idx_hbm, const_hbm, out_hbm,
                 idx_all, rows_a, rows_b, const_v,
                 gsem_a, gsem_b, wsem_a, wsem_b):
    wid = lax.axis_index("s") * 2 + lax.axis_index("c")
    pltpu.sync_copy(const_hbm, const_v)

    row_bufs = (rows_a, rows_b)
    gsems = (gsem_a, gsem_b)
    wsems = (wsem_a, wsem_b)

    def fire_gather(j, buf):
        c = wid * _CPW + j
        pltpu.sync_copy(idx_hbm.at[pl.ds(2 * c, 2)], idx_all.at[buf])
        cps = []
        for h in range(2):
            cps.append(pltpu.async_copy(
                table_hbm.at[idx_all.at[buf].at[h]],
                row_bufs[buf].at[pl.ds(h * 128, 128)],
                gsems[buf]))
        return cps

    def add_const(j, buf):
        c = wid * _CPW + j
        rows = row_bufs[buf]
        s0 = lax.rem(c * _CHUNK, SEQ)

        def add_row(t, carry):
            vals = [const_v[s0 + t, pl.ds(l * 16, 16)] for l in range(HID // 16)]
            for l in range(HID // 16):
                plsc.addupdate(rows.at[t, pl.ds(l * 16, 16)], vals[l])
            return carry

        lax.fori_loop(0, _CHUNK, add_row, 0)

    g_pend = {0: fire_gather(0, 0)}
    w_pend = {}
    for j in range(_CPW):
        buf = j & 1
        nxt = buf ^ 1
        if j + 1 < _CPW:
            if (j - 1) in w_pend:
                w_pend.pop(j - 1).wait()
            g_pend[j + 1] = fire_gather(j + 1, nxt)
        for cp in g_pend.pop(j):
            cp.wait()
        add_const(j, buf)
        c = wid * _CPW + j
        w_pend[j] = pltpu.async_copy(
            row_bufs[buf], out_hbm.at[pl.ds(c * _CHUNK, _CHUNK)], wsems[buf])
    for d in w_pend.values():
        d.wait()


def _gather(table2, idx2, const2):
    mesh = plsc.VectorSubcoreMesh(core_axis_name="c", subcore_axis_name="s")
    k = functools.partial(
        pl.kernel,
        mesh=mesh,
        out_type=jax.ShapeDtypeStruct((NFLAT, HID), jnp.float32),
        scratch_types=[
            pltpu.VMEM((2, 2, 128), jnp.int32),
            pltpu.VMEM((_CHUNK, HID), jnp.float32),
            pltpu.VMEM((_CHUNK, HID), jnp.float32),
            pltpu.VMEM((_CEXT, HID), jnp.float32),
            pltpu.SemaphoreType.DMA,
            pltpu.SemaphoreType.DMA,
            pltpu.SemaphoreType.DMA,
            pltpu.SemaphoreType.DMA,
        ],
    )(_gather_body)
    return k(table2, idx2, const2)


def kernel(inputs, word_table, pos_table, W, b, gamma, beta, moving_mean, moving_var):
    scale = gamma * lax.rsqrt(moving_var + 1e-3)
    bias2 = (b - moving_mean) * scale + beta
    table2, const = _transform(
        pos_table, word_table, W, scale[None, :], bias2[None, :]
    )
    idx2 = inputs.reshape(NFLAT // 128, 128).astype(jnp.int32)
    out4 = _gather(table2, idx2, const)
    return out4.reshape(BATCH, SEQ, HID)


# speedup vs baseline: 5.2323x; 1.0088x over previous
import functools

import jax
import jax.numpy as jnp
from jax import lax
from jax.experimental import pallas as pl
from jax.experimental.pallas import tpu as pltpu
from jax.experimental.pallas import tpu_sc as plsc

HID = 128
POS = 100000
BATCH = 1024
SEQ = 200
NFLAT = BATCH * SEQ

_ROWS = 20000
_GRID = POS // _ROWS


_CEXT = 456


def _transform_body(pos_ref, word_ref, w_ref, b_ref, gamma_ref, beta_ref,
                    mu_ref, var_ref, t2_ref, const_ref):
    scale = gamma_ref[...] * lax.rsqrt(var_ref[...] + 1e-3)
    ws = w_ref[...] * scale
    t2_ref[...] = jnp.dot(pos_ref[...], ws, preferred_element_type=jnp.float32)

    @pl.when(pl.program_id(0) == 0)
    def _():
        bias2 = (b_ref[...] - mu_ref[...]) * scale + beta_ref[...]
        cmat = (
            jnp.dot(word_ref[...], ws, preferred_element_type=jnp.float32)
            + bias2
        )
        const_ref[0:SEQ, :] = cmat
        const_ref[SEQ : 2 * SEQ, :] = cmat
        const_ref[2 * SEQ : _CEXT, :] = cmat[: _CEXT - 2 * SEQ, :]


def _transform(pos_table, word_table, W, b, gamma, beta, mu, var):
    vec = pl.BlockSpec((1, HID), lambda i: (0, 0))
    return pl.pallas_call(
        _transform_body,
        grid=(_GRID,),
        in_specs=[
            pl.BlockSpec((_ROWS, HID), lambda i: (i, 0)),
            pl.BlockSpec((SEQ, HID), lambda i: (0, 0)),
            pl.BlockSpec((HID, HID), lambda i: (0, 0)),
            vec, vec, vec, vec, vec,
        ],
        out_specs=[
            pl.BlockSpec((_ROWS, HID), lambda i: (i, 0)),
            pl.BlockSpec((_CEXT, HID), lambda i: (0, 0)),
        ],
        out_shape=[
            jax.ShapeDtypeStruct((POS, HID), jnp.float32),
            jax.ShapeDtypeStruct((_CEXT, HID), jnp.float32),
        ],
    )(pos_table, word_table, W, b, gamma, beta, mu, var)


_NW = 32
_CHUNK = 256
_NCHUNK = NFLAT // _CHUNK
_CPW = _NCHUNK // _NW


def _gather_body(table_hbm, idx_hbm, const_hbm, out_hbm,
                 idx_all, rows_a, rows_b, const_v,
                 gsem_a, gsem_b, wsem_a, wsem_b):
    wid = lax.axis_index("s") * 2 + lax.axis_index("c")
    pltpu.sync_copy(const_hbm, const_v)

    row_bufs = (rows_a, rows_b)
    gsems = (gsem_a, gsem_b)
    wsems = (wsem_a, wsem_b)

    def fire_gather(j, buf):
        c = wid * _CPW + j
        pltpu.sync_copy(idx_hbm.at[pl.ds(2 * c, 2)], idx_all.at[buf])
        cps = []
        for h in range(2):
            cps.append(pltpu.async_copy(
                table_hbm.at[idx_all.at[buf].at[h]],
                row_bufs[buf].at[pl.ds(h * 128, 128)],
                gsems[buf]))
        return cps

    def add_const(j, buf):
        c = wid * _CPW + j
        rows = row_bufs[buf]
        s0 = lax.rem(c * _CHUNK, SEQ)

        def add_row(t, carry):
            vals = [const_v[s0 + t, pl.ds(l * 16, 16)] for l in range(HID // 16)]
            for l in range(HID // 16):
                plsc.addupdate(rows.at[t, pl.ds(l * 16, 16)], vals[l])
            return carry

        lax.fori_loop(0, _CHUNK, add_row, 0)

    g_pend = {0: fire_gather(0, 0)}
    w_pend = {}
    for j in range(_CPW):
        buf = j & 1
        nxt = buf ^ 1
        if j + 1 < _CPW:
            if (j - 1) in w_pend:
                w_pend.pop(j - 1).wait()
            g_pend[j + 1] = fire_gather(j + 1, nxt)
        for cp in g_pend.pop(j):
            cp.wait()
        add_const(j, buf)
        c = wid * _CPW + j
        w_pend[j] = pltpu.async_copy(
            row_bufs[buf], out_hbm.at[pl.ds(c * _CHUNK, _CHUNK)], wsems[buf])
    for d in w_pend.values():
        d.wait()


def _gather(table2, idx2, const2):
    mesh = plsc.VectorSubcoreMesh(core_axis_name="c", subcore_axis_name="s")
    k = functools.partial(
        pl.kernel,
        mesh=mesh,
        out_type=jax.ShapeDtypeStruct((NFLAT, HID), jnp.float32),
        scratch_types=[
            pltpu.VMEM((2, 2, 128), jnp.int32),
            pltpu.VMEM((_CHUNK, HID), jnp.float32),
            pltpu.VMEM((_CHUNK, HID), jnp.float32),
            pltpu.VMEM((_CEXT, HID), jnp.float32),
            pltpu.SemaphoreType.DMA,
            pltpu.SemaphoreType.DMA,
            pltpu.SemaphoreType.DMA,
            pltpu.SemaphoreType.DMA,
        ],
    )(_gather_body)
    return k(table2, idx2, const2)


def kernel(inputs, word_table, pos_table, W, b, gamma, beta, moving_mean, moving_var):
    table2, const = _transform(
        pos_table, word_table, W, b[None, :], gamma[None, :], beta[None, :],
        moving_mean[None, :], moving_var[None, :]
    )
    idx2 = inputs.reshape(NFLAT // 128, 128).astype(jnp.int32)
    out4 = _gather(table2, idx2, const)
    return out4.reshape(BATCH, SEQ, HID)
